# Initial kernel scaffold; baseline (speedup 1.0000x reference)
import jax
import jax.numpy as jnp
from jax.experimental import pallas as pl


def kernel(indices_expert, weight1, weight2, device_indices_map, local_expert_indices_map):
    raise NotImplementedError("write your pallas kernel here")


# speedup vs baseline: 1.3024x; 1.3024x over previous
import functools

import jax
import jax.numpy as jnp
from jax import lax
from jax.experimental import pallas as pl
from jax.experimental.pallas import tpu as pltpu
from jax.experimental.pallas import tpu_sc as plsc

_NC = 2
_NS = 16
_NW = _NC * _NS
_L = 16
_MAP = 64


def _build(n):
    per_w = n // _NW
    mesh = plsc.VectorSubcoreMesh(
        core_axis_name="c", subcore_axis_name="s",
        num_cores=_NC, num_subcores=_NS)

    @functools.partial(
        pl.kernel,
        out_type=(jax.ShapeDtypeStruct((n,), jnp.int32),
                  jax.ShapeDtypeStruct((n,), jnp.int32)),
        mesh=mesh,
        compiler_params=pltpu.CompilerParams(needs_layout_passes=False),
        scratch_types=[
            pltpu.VMEM((per_w,), jnp.int32),
            pltpu.VMEM((per_w,), jnp.int32),
            pltpu.VMEM((per_w,), jnp.int32),
            pltpu.VMEM((128,), jnp.int32),
            pltpu.VMEM((128,), jnp.int32),
        ],
    )
    def dispatch(idx_hbm, devmap_hbm, locmap_hbm, dev_hbm, loc_hbm,
                 idx_v, dev_v, loc_v, devmap_v, locmap_v):
        wid = lax.axis_index("s") * _NC + lax.axis_index("c")
        base = wid * per_w
        pltpu.sync_copy(devmap_hbm, devmap_v.at[pl.ds(0, _MAP)])
        pltpu.sync_copy(locmap_hbm, locmap_v.at[pl.ds(0, _MAP)])
        pltpu.sync_copy(idx_hbm.at[pl.ds(base, per_w)], idx_v)

        def step(i, _):
            iv = idx_v[pl.ds(i * _L, _L)]
            dev_v[pl.ds(i * _L, _L)] = plsc.load_gather(devmap_v, [iv])
            loc_v[pl.ds(i * _L, _L)] = plsc.load_gather(locmap_v, [iv])
            return 0

        lax.fori_loop(0, per_w // _L, step, 0, unroll=8)
        pltpu.sync_copy(dev_v, dev_hbm.at[pl.ds(base, per_w)])
        pltpu.sync_copy(loc_v, loc_hbm.at[pl.ds(base, per_w)])

    return dispatch


def kernel(indices_expert, weight1, weight2, device_indices_map,
           local_expert_indices_map):
    t, k = indices_expert.shape
    n = t * k
    flat = indices_expert.reshape(n).astype(jnp.int32)
    dev, loc = _build(n)(flat,
                         device_indices_map.astype(jnp.int32),
                         local_expert_indices_map.astype(jnp.int32))
    out_dtype = indices_expert.dtype
    return (dev.reshape(t, k).astype(out_dtype),
            loc.reshape(t, k).astype(out_dtype))


# speedup vs baseline: 1.3279x; 1.0196x over previous
import functools

import jax
import jax.numpy as jnp
from jax import lax
from jax.experimental import pallas as pl
from jax.experimental.pallas import tpu as pltpu
from jax.experimental.pallas import tpu_sc as plsc

_NC = 2
_NS = 16
_NW = _NC * _NS
_L = 16
_MAP = 64


def _build(n):
    per_w = n // _NW
    mesh = plsc.VectorSubcoreMesh(
        core_axis_name="c", subcore_axis_name="s",
        num_cores=_NC, num_subcores=_NS)

    @functools.partial(
        pl.kernel,
        out_type=(jax.ShapeDtypeStruct((n,), jnp.int32),
                  jax.ShapeDtypeStruct((n,), jnp.int32)),
        mesh=mesh,
        compiler_params=pltpu.CompilerParams(
            needs_layout_passes=False,
            disable_bounds_checks=True,
            disable_semaphore_checks=True,
            skip_device_barrier=True),
        scratch_types=[
            pltpu.VMEM((per_w,), jnp.int32),
            pltpu.VMEM((per_w,), jnp.int32),
            pltpu.VMEM((per_w,), jnp.int32),
            pltpu.VMEM((128,), jnp.int32),
            pltpu.VMEM((128,), jnp.int32),
            pltpu.VMEM((128,), jnp.int32),
        ],
    )
    def dispatch(idx_hbm, devmap_hbm, locmap_hbm, dev_hbm, loc_hbm,
                 idx_v, dev_v, loc_v, devmap_v, locmap_v, packed_v):
        wid = lax.axis_index("s") * _NC + lax.axis_index("c")
        base = wid * per_w
        pltpu.sync_copy(devmap_hbm, devmap_v.at[pl.ds(0, _MAP)])
        pltpu.sync_copy(locmap_hbm, locmap_v.at[pl.ds(0, _MAP)])
        pltpu.sync_copy(idx_hbm.at[pl.ds(base, per_w)], idx_v)

        for j in range(_MAP // _L):
            sl = pl.ds(j * _L, _L)
            packed_v[sl] = (devmap_v[sl] << 16) | (locmap_v[sl] & 0xFFFF)

        def step(i, _):
            sl = pl.ds(i * _L, _L)
            g = plsc.load_gather(packed_v, [idx_v[sl]])
            dev_v[sl] = g >> 16
            loc_v[sl] = (g << 16) >> 16
            return 0

        lax.fori_loop(0, per_w // _L, step, 0, unroll=8)
        pltpu.sync_copy(dev_v, dev_hbm.at[pl.ds(base, per_w)])
        pltpu.sync_copy(loc_v, loc_hbm.at[pl.ds(base, per_w)])

    return dispatch


def kernel(indices_expert, weight1, weight2, device_indices_map,
           local_expert_indices_map):
    t, k = indices_expert.shape
    n = t * k
    flat = indices_expert.reshape(n).astype(jnp.int32)
    dev, loc = _build(n)(flat,
                         device_indices_map.astype(jnp.int32),
                         local_expert_indices_map.astype(jnp.int32))
    out_dtype = indices_expert.dtype
    return (dev.reshape(t, k).astype(out_dtype),
            loc.reshape(t, k).astype(out_dtype))
